# Initial kernel scaffold; baseline (speedup 1.0000x reference)
import jax
import jax.numpy as jnp
from jax.experimental import pallas as pl


def kernel(inputs, weights):
    raise NotImplementedError("write your pallas kernel here")


# speedup vs baseline: 2.1342x; 2.1342x over previous
import functools

import jax
import jax.numpy as jnp
from jax import lax
from jax.experimental import pallas as pl
from jax.experimental.pallas import tpu as pltpu
from jax.experimental.pallas import tpu_sc as plsc

PAD = 1
BATCH = 4
SEQ = 8192
DIM = 1024
NCORES = 2
NSUB = 16
NWORK = NCORES * NSUB
CHUNK = (BATCH * SEQ) // NWORK
ROWCHUNKS = SEQ // CHUNK
K = 32
NSTEP = CHUNK // K
VPC = CHUNK // 16

_mesh = plsc.VectorSubcoreMesh(core_axis_name="c", subcore_axis_name="s")


@functools.partial(
    pl.kernel,
    mesh=_mesh,
    out_type=jax.ShapeDtypeStruct((BATCH * SEQ, DIM), jnp.float32),
    compiler_params=pltpu.CompilerParams(needs_layout_passes=False),
    scratch_types=[
        pltpu.VMEM((SEQ,), jnp.int32),
        pltpu.VMEM((NSTEP, K), jnp.int32),
        pltpu.VMEM((K, DIM), jnp.float32),
        pltpu.VMEM((K, DIM), jnp.float32),
        pltpu.SemaphoreType.DMA,
        pltpu.SemaphoreType.DMA,
        pltpu.SemaphoreType.DMA,
        pltpu.SemaphoreType.DMA,
    ],
)
def _pos_embed_sc(tok_hbm, w_hbm, out_hbm, row_v, pos_v, buf0, buf1,
                  g0, g1, o0, o1):
    cid = lax.axis_index("c")
    sid = lax.axis_index("s")
    wid = cid * NSUB + sid
    b = wid // ROWCHUNKS
    cpos = wid % ROWCHUNKS

    pltpu.sync_copy(tok_hbm.at[b], row_v)

    def count_body(i, acc):
        v = row_v[pl.ds(i * 16, 16)]
        return acc + jnp.minimum(jnp.abs(v - PAD), 1)

    acc = lax.fori_loop(0, cpos * (CHUNK // 16), count_body,
                        jnp.zeros((16,), jnp.int32))
    carry = jnp.sum(acc)

    base = cpos * CHUNK
    for i in range(VPC):
        v = row_v[pl.ds(base + i * 16, 16)]
        m = jnp.minimum(jnp.abs(v - PAD), 1)
        cum = jnp.cumsum(m) + carry
        pos_v[i // 2, pl.ds((i % 2) * 16, 16)] = cum * m + PAD
        carry = carry + jnp.sum(m)

    bufs = (buf0, buf1)
    gsem = (g0, g1)
    osem = (o0, o1)
    rbase = wid * CHUNK
    gh = [None] * NSTEP
    oh = [None] * NSTEP
    gh[0] = pltpu.async_copy(w_hbm.at[pos_v.at[0]], bufs[0], gsem[0])
    for t in range(NSTEP):
        pb = t % 2
        gh[t].wait()
        oh[t] = pltpu.async_copy(
            bufs[pb], out_hbm.at[pl.ds(rbase + t * K, K)], osem[pb])
        if t + 1 < NSTEP:
            if t >= 1:
                oh[t - 1].wait()
            gh[t + 1] = pltpu.async_copy(
                w_hbm.at[pos_v.at[t + 1]], bufs[(t + 1) % 2],
                gsem[(t + 1) % 2])
    oh[NSTEP - 2].wait()
    oh[NSTEP - 1].wait()


def kernel(inputs, weights):
    tok = inputs.astype(jnp.int32)
    out = _pos_embed_sc(tok, weights)
    return lax.stop_gradient(out.reshape(BATCH, SEQ, DIM))


# speedup vs baseline: 2.2313x; 1.0455x over previous
import functools

import jax
import jax.numpy as jnp
from jax import lax
from jax.experimental import pallas as pl
from jax.experimental.pallas import tpu as pltpu
from jax.experimental.pallas import tpu_sc as plsc

PAD = 1
BATCH = 4
SEQ = 8192
DIM = 1024
NCORES = 2
NSUB = 16
NWORK = NCORES * NSUB
CHUNK = (BATCH * SEQ) // NWORK
ROWCHUNKS = SEQ // CHUNK
K = 32
NSTEP = CHUNK // K
VPS = K // 16
NBUF = 3

_mesh = plsc.VectorSubcoreMesh(core_axis_name="c", subcore_axis_name="s")


@functools.partial(
    pl.kernel,
    mesh=_mesh,
    out_type=jax.ShapeDtypeStruct((BATCH * SEQ, DIM), jnp.float32),
    compiler_params=pltpu.CompilerParams(needs_layout_passes=False),
    scratch_types=[
        pltpu.VMEM((SEQ,), jnp.int32),
        pltpu.VMEM((NSTEP, K), jnp.int32),
        pltpu.VMEM((K, DIM), jnp.float32),
        pltpu.VMEM((K, DIM), jnp.float32),
        pltpu.VMEM((K, DIM), jnp.float32),
        pltpu.SemaphoreType.DMA,
        pltpu.SemaphoreType.DMA,
        pltpu.SemaphoreType.DMA,
        pltpu.SemaphoreType.DMA,
        pltpu.SemaphoreType.DMA,
        pltpu.SemaphoreType.DMA,
    ],
)
def _pos_embed_sc(tok_hbm, w_hbm, out_hbm, row_v, pos_v, buf0, buf1, buf2,
                  g0, g1, g2, o0, o1, o2):
    cid = lax.axis_index("c")
    sid = lax.axis_index("s")
    wid = cid * NSUB + sid
    b = wid // ROWCHUNKS
    cpos = wid % ROWCHUNKS

    pltpu.sync_copy(tok_hbm.at[b], row_v)

    def count_body(i, acc):
        for u in range(4):
            v = row_v[pl.ds(i * 64 + u * 16, 16)]
            acc = acc + jnp.minimum(jnp.abs(v - PAD), 1)
        return acc

    acc = lax.fori_loop(0, cpos * (CHUNK // 64), count_body,
                        jnp.zeros((16,), jnp.int32))
    carry = jnp.sum(acc)

    base = cpos * CHUNK

    def compute_step(s, carry):
        for q in range(VPS):
            v = row_v[pl.ds(base + (s * VPS + q) * 16, 16)]
            m = jnp.minimum(jnp.abs(v - PAD), 1)
            cum = jnp.cumsum(m) + carry
            pos_v[s, pl.ds(q * 16, 16)] = cum * m + PAD
            carry = carry + jnp.sum(m)
        return carry

    bufs = (buf0, buf1, buf2)
    gsem = (g0, g1, g2)
    osem = (o0, o1, o2)
    rbase = wid * CHUNK

    def gather(t):
        return pltpu.async_copy(
            w_hbm.at[pos_v.at[t]], bufs[t % NBUF], gsem[t % NBUF])

    def writeout(t):
        return pltpu.async_copy(
            bufs[t % NBUF], out_hbm.at[pl.ds(rbase + t * K, K)],
            osem[t % NBUF])

    gh = [None] * NSTEP
    oh = [None] * NSTEP
    carry = compute_step(0, carry)
    carry = compute_step(1, carry)
    gh[0] = gather(0)
    gh[1] = gather(1)
    for t in range(NSTEP):
        if t + 2 < NSTEP:
            carry = compute_step(t + 2, carry)
        gh[t].wait()
        oh[t] = writeout(t)
        if t + 2 < NSTEP:
            if t >= 1:
                oh[t - 1].wait()
            gh[t + 2] = gather(t + 2)
    oh[NSTEP - 2].wait()
    oh[NSTEP - 1].wait()


def kernel(inputs, weights):
    tok = inputs.astype(jnp.int32)
    out = _pos_embed_sc(tok, weights)
    return lax.stop_gradient(out.reshape(BATCH, SEQ, DIM))


# speedup vs baseline: 2.2464x; 1.0068x over previous
import functools

import jax
import jax.numpy as jnp
from jax import lax
from jax.experimental import pallas as pl
from jax.experimental.pallas import tpu as pltpu
from jax.experimental.pallas import tpu_sc as plsc

PAD = 1
BATCH = 4
SEQ = 8192
DIM = 1024
NCORES = 2
NSUB = 16
NWORK = NCORES * NSUB
CHUNK = (BATCH * SEQ) // NWORK
ROWCHUNKS = SEQ // CHUNK
K = 16
NSTEP = CHUNK // K
VPS = K // 16
NBUF = 6
NFLY = 3

_mesh = plsc.VectorSubcoreMesh(core_axis_name="c", subcore_axis_name="s")


@functools.partial(
    pl.kernel,
    mesh=_mesh,
    out_type=jax.ShapeDtypeStruct((BATCH * SEQ, DIM), jnp.float32),
    compiler_params=pltpu.CompilerParams(needs_layout_passes=False),
    scratch_types=[
        pltpu.VMEM((SEQ,), jnp.int32),
        pltpu.VMEM((NSTEP, K), jnp.int32),
    ]
    + [pltpu.VMEM((K, DIM), jnp.float32)] * NBUF
    + [pltpu.SemaphoreType.DMA] * NBUF
    + [pltpu.SemaphoreType.DMA] * NBUF,
)
def _pos_embed_sc(tok_hbm, w_hbm, out_hbm, row_v, pos_v, *rest):
    bufs = rest[:NBUF]
    gsem = rest[NBUF:2 * NBUF]
    osem = rest[2 * NBUF:3 * NBUF]
    cid = lax.axis_index("c")
    sid = lax.axis_index("s")
    wid = cid * NSUB + sid
    b = wid // ROWCHUNKS
    cpos = wid % ROWCHUNKS

    pltpu.sync_copy(tok_hbm.at[b], row_v)

    def count_body(i, acc):
        for u in range(4):
            v = row_v[pl.ds(i * 64 + u * 16, 16)]
            acc = acc + jnp.minimum(jnp.abs(v - PAD), 1)
        return acc

    acc = lax.fori_loop(0, cpos * (CHUNK // 64), count_body,
                        jnp.zeros((16,), jnp.int32))
    carry = jnp.sum(acc)

    base = cpos * CHUNK

    def compute_step(s, carry):
        for q in range(VPS):
            v = row_v[pl.ds(base + (s * VPS + q) * 16, 16)]
            m = jnp.minimum(jnp.abs(v - PAD), 1)
            cum = jnp.cumsum(m) + carry
            pos_v[s, pl.ds(q * 16, 16)] = cum * m + PAD
            carry = carry + jnp.sum(m)
        return carry

    rbase = wid * CHUNK

    def gather(t):
        return pltpu.async_copy(
            w_hbm.at[pos_v.at[t]], bufs[t % NBUF], gsem[t % NBUF])

    def writeout(t):
        return pltpu.async_copy(
            bufs[t % NBUF], out_hbm.at[pl.ds(rbase + t * K, K)],
            osem[t % NBUF])

    gh = [None] * NSTEP
    oh = [None] * NSTEP
    for t in range(NFLY):
        carry = compute_step(t, carry)
        gh[t] = gather(t)
    for t in range(NSTEP):
        if t + NFLY < NSTEP:
            carry = compute_step(t + NFLY, carry)
        gh[t].wait()
        oh[t] = writeout(t)
        if t + NFLY < NSTEP:
            tb = t + NFLY - NBUF
            if tb >= 0:
                oh[tb].wait()
            gh[t + NFLY] = gather(t + NFLY)
    for t in range(max(0, NSTEP - NBUF), NSTEP):
        oh[t].wait()


def kernel(inputs, weights):
    tok = inputs.astype(jnp.int32)
    out = _pos_embed_sc(tok, weights)
    return lax.stop_gradient(out.reshape(BATCH, SEQ, DIM))


# speedup vs baseline: 2.2518x; 1.0024x over previous
import functools

import jax
import jax.numpy as jnp
from jax import lax
from jax.experimental import pallas as pl
from jax.experimental.pallas import tpu as pltpu
from jax.experimental.pallas import tpu_sc as plsc

PAD = 1
BATCH = 4
SEQ = 8192
DIM = 1024
NCORES = 2
NSUB = 16
NWORK = NCORES * NSUB
CHUNK = (BATCH * SEQ) // NWORK
ROWCHUNKS = SEQ // CHUNK
K = 16
NSTEP = CHUNK // K
VPS = K // 16
NBUF = 6
NFLY = 5

_mesh = plsc.VectorSubcoreMesh(core_axis_name="c", subcore_axis_name="s")


@functools.partial(
    pl.kernel,
    mesh=_mesh,
    out_type=jax.ShapeDtypeStruct((BATCH * SEQ, DIM), jnp.float32),
    compiler_params=pltpu.CompilerParams(needs_layout_passes=False),
    scratch_types=[
        pltpu.VMEM((SEQ,), jnp.int32),
        pltpu.VMEM((NSTEP, K), jnp.int32),
    ]
    + [pltpu.VMEM((K, DIM), jnp.float32)] * NBUF
    + [pltpu.SemaphoreType.DMA] * NBUF
    + [pltpu.SemaphoreType.DMA] * NBUF,
)
def _pos_embed_sc(tok_hbm, w_hbm, out_hbm, row_v, pos_v, *rest):
    bufs = rest[:NBUF]
    gsem = rest[NBUF:2 * NBUF]
    osem = rest[2 * NBUF:3 * NBUF]
    cid = lax.axis_index("c")
    sid = lax.axis_index("s")
    wid = cid * NSUB + sid
    b = wid // ROWCHUNKS
    cpos = wid % ROWCHUNKS

    pltpu.sync_copy(tok_hbm.at[b], row_v)

    def count_body(i, acc):
        for u in range(4):
            v = row_v[pl.ds(i * 64 + u * 16, 16)]
            acc = acc + jnp.minimum(jnp.abs(v - PAD), 1)
        return acc

    acc = lax.fori_loop(0, cpos * (CHUNK // 64), count_body,
                        jnp.zeros((16,), jnp.int32))
    carry = jnp.sum(acc)

    base = cpos * CHUNK

    def compute_step(s, carry):
        for q in range(VPS):
            v = row_v[pl.ds(base + (s * VPS + q) * 16, 16)]
            m = jnp.minimum(jnp.abs(v - PAD), 1)
            cum = jnp.cumsum(m) + carry
            pos_v[s, pl.ds(q * 16, 16)] = cum * m + PAD
            carry = carry + jnp.sum(m)
        return carry

    rbase = wid * CHUNK

    def gather(t):
        return pltpu.async_copy(
            w_hbm.at[pos_v.at[t]], bufs[t % NBUF], gsem[t % NBUF])

    def writeout(t):
        return pltpu.async_copy(
            bufs[t % NBUF], out_hbm.at[pl.ds(rbase + t * K, K)],
            osem[t % NBUF])

    gh = [None] * NSTEP
    oh = [None] * NSTEP
    for t in range(NFLY):
        carry = compute_step(t, carry)
        gh[t] = gather(t)
    for t in range(NSTEP):
        if t + NFLY < NSTEP:
            carry = compute_step(t + NFLY, carry)
        gh[t].wait()
        oh[t] = writeout(t)
        if t + NFLY < NSTEP:
            tb = t + NFLY - NBUF
            if tb >= 0:
                oh[tb].wait()
            gh[t + NFLY] = gather(t + NFLY)
    for t in range(max(0, NSTEP - NBUF), NSTEP):
        oh[t].wait()


def kernel(inputs, weights):
    tok = inputs.astype(jnp.int32)
    out = _pos_embed_sc(tok, weights)
    return lax.stop_gradient(out.reshape(BATCH, SEQ, DIM))


# speedup vs baseline: 2.2546x; 1.0013x over previous
import functools

import jax
import jax.numpy as jnp
from jax import lax
from jax.experimental import pallas as pl
from jax.experimental.pallas import tpu as pltpu
from jax.experimental.pallas import tpu_sc as plsc

PAD = 1
BATCH = 4
SEQ = 8192
DIM = 1024
NCORES = 2
NSUB = 16
NWORK = NCORES * NSUB
CHUNK = (BATCH * SEQ) // NWORK
ROWCHUNKS = SEQ // CHUNK
K = 16
NSTEP = CHUNK // K
VPS = K // 16
NBUF = 6
NFLY = 5

_mesh = plsc.VectorSubcoreMesh(core_axis_name="c", subcore_axis_name="s")


@functools.partial(
    pl.kernel,
    mesh=_mesh,
    out_type=jax.ShapeDtypeStruct((BATCH * SEQ, DIM), jnp.float32),
    compiler_params=pltpu.CompilerParams(needs_layout_passes=False),
    scratch_types=[
        pltpu.VMEM((SEQ,), jnp.int32),
        pltpu.VMEM((NSTEP, K), jnp.int32),
    ]
    + [pltpu.VMEM((K, DIM), jnp.float32)] * NBUF
    + [pltpu.SemaphoreType.DMA] * NBUF
    + [pltpu.SemaphoreType.DMA] * NBUF,
)
def _pos_embed_sc(tok_hbm, w_hbm, out_hbm, row_v, pos_v, *rest):
    bufs = rest[:NBUF]
    gsem = rest[NBUF:2 * NBUF]
    osem = rest[2 * NBUF:3 * NBUF]
    cid = lax.axis_index("c")
    sid = lax.axis_index("s")
    wid = cid * NSUB + sid
    b = wid // ROWCHUNKS
    cpos = wid % ROWCHUNKS

    pltpu.sync_copy(tok_hbm.at[b], row_v)

    def count_body(i, acc):
        for u in range(8):
            v = row_v[pl.ds(i * 128 + u * 16, 16)]
            acc = acc + jnp.minimum(jnp.abs(v - PAD), 1)
        return acc

    acc = lax.fori_loop(0, cpos * (CHUNK // 128), count_body,
                        jnp.zeros((16,), jnp.int32))
    carry = jnp.sum(acc)

    base = cpos * CHUNK

    def compute_step(s, carry):
        for q in range(VPS):
            v = row_v[pl.ds(base + (s * VPS + q) * 16, 16)]
            m = jnp.minimum(jnp.abs(v - PAD), 1)
            cum = jnp.cumsum(m) + carry
            pos_v[s, pl.ds(q * 16, 16)] = cum * m + PAD
            carry = carry + jnp.sum(m)
        return carry

    rbase = wid * CHUNK

    def gather(t):
        return pltpu.async_copy(
            w_hbm.at[pos_v.at[t]], bufs[t % NBUF], gsem[t % NBUF])

    def writeout(t):
        return pltpu.async_copy(
            bufs[t % NBUF], out_hbm.at[pl.ds(rbase + t * K, K)],
            osem[t % NBUF])

    gh = [None] * NSTEP
    oh = [None] * NSTEP
    for t in range(NFLY):
        carry = compute_step(t, carry)
        gh[t] = gather(t)
    for t in range(NSTEP):
        if t + NFLY < NSTEP:
            carry = compute_step(t + NFLY, carry)
        gh[t].wait()
        oh[t] = writeout(t)
        if t + NFLY < NSTEP:
            tb = t + NFLY - NBUF
            if tb >= 0:
                oh[tb].wait()
            gh[t + NFLY] = gather(t + NFLY)
    for t in range(max(0, NSTEP - NBUF), NSTEP):
        oh[t].wait()


def kernel(inputs, weights):
    tok = inputs.astype(jnp.int32)
    out = _pos_embed_sc(tok, weights)
    return lax.stop_gradient(out.reshape(BATCH, SEQ, DIM))


# speedup vs baseline: 2.2636x; 1.0040x over previous
import functools

import jax
import jax.numpy as jnp
from jax import lax
from jax.experimental import pallas as pl
from jax.experimental.pallas import tpu as pltpu
from jax.experimental.pallas import tpu_sc as plsc

PAD = 1
BATCH = 4
SEQ = 8192
DIM = 1024
NCORES = 2
NSUB = 16
NWORK = NCORES * NSUB
CHUNK = (BATCH * SEQ) // NWORK
ROWCHUNKS = SEQ // CHUNK
K = 16
NSTEP = CHUNK // K
VPS = K // 16
NBUF = 6
NFLY = 5

_mesh = plsc.VectorSubcoreMesh(core_axis_name="c", subcore_axis_name="s")


@functools.partial(
    pl.kernel,
    mesh=_mesh,
    out_type=jax.ShapeDtypeStruct((BATCH * SEQ, DIM), jnp.float32),
    compiler_params=pltpu.CompilerParams(needs_layout_passes=False),
    scratch_types=[
        pltpu.VMEM((SEQ,), jnp.int32),
    ]
    + [pltpu.VMEM((K, DIM), jnp.float32)] * NBUF
    + [pltpu.SemaphoreType.DMA] * NBUF
    + [pltpu.SemaphoreType.DMA] * NBUF,
)
def _pos_embed_sc(tok_hbm, w_hbm, out_hbm, row_v, *rest):
    bufs = rest[:NBUF]
    gsem = rest[NBUF:2 * NBUF]
    osem = rest[2 * NBUF:3 * NBUF]
    cid = lax.axis_index("c")
    sid = lax.axis_index("s")
    wid = cid * NSUB + sid
    b = wid // ROWCHUNKS
    cpos = wid % ROWCHUNKS

    pltpu.sync_copy(tok_hbm.at[b], row_v)

    def count_body(i, acc):
        for u in range(8):
            v = row_v[pl.ds(i * 128 + u * 16, 16)]
            acc = acc + jnp.minimum(jnp.abs(v - PAD), 1)
        return acc

    acc = lax.fori_loop(0, cpos * (CHUNK // 128), count_body,
                        jnp.zeros((16,), jnp.int32))
    carry = jnp.sum(acc)

    base = cpos * CHUNK

    def compute_step(s, carry):
        v = row_v[pl.ds(base + s * 16, 16)]
        m = jnp.minimum(jnp.abs(v - PAD), 1)
        cum = jnp.cumsum(m) + carry
        pos = cum * m + PAD
        return pos, carry + jnp.sum(m)

    rbase = wid * CHUNK

    def gather(t, pos):
        return pltpu.async_copy(
            w_hbm.at[pos], bufs[t % NBUF], gsem[t % NBUF])

    def writeout(t):
        return pltpu.async_copy(
            bufs[t % NBUF], out_hbm.at[pl.ds(rbase + t * K, K)],
            osem[t % NBUF])

    gh = [None] * NSTEP
    oh = [None] * NSTEP
    for t in range(NFLY):
        pos, carry = compute_step(t, carry)
        gh[t] = gather(t, pos)
    for t in range(NSTEP):
        if t + NFLY < NSTEP:
            pos, carry = compute_step(t + NFLY, carry)
        gh[t].wait()
        oh[t] = writeout(t)
        if t + NFLY < NSTEP:
            tb = t + NFLY - NBUF
            if tb >= 0:
                oh[tb].wait()
            gh[t + NFLY] = gather(t + NFLY, pos)
    for t in range(max(0, NSTEP - NBUF), NSTEP):
        oh[t].wait()


def kernel(inputs, weights):
    tok = inputs.astype(jnp.int32)
    out = _pos_embed_sc(tok, weights)
    return lax.stop_gradient(out.reshape(BATCH, SEQ, DIM))


# speedup vs baseline: 2.2655x; 1.0008x over previous
import functools

import jax
import jax.numpy as jnp
from jax import lax
from jax.experimental import pallas as pl
from jax.experimental.pallas import tpu as pltpu
from jax.experimental.pallas import tpu_sc as plsc

PAD = 1
BATCH = 4
SEQ = 8192
DIM = 1024
NCORES = 2
NSUB = 16
NWORK = NCORES * NSUB
CHUNK = (BATCH * SEQ) // NWORK
ROWCHUNKS = SEQ // CHUNK
K = 16
NSTEP = CHUNK // K
NBUF = 6
NFLY = 5

_mesh = plsc.VectorSubcoreMesh(core_axis_name="c", subcore_axis_name="s")


@functools.partial(
    pl.kernel,
    mesh=_mesh,
    out_type=jax.ShapeDtypeStruct((BATCH * SEQ, DIM), jnp.float32),
    compiler_params=pltpu.CompilerParams(needs_layout_passes=False),
    scratch_types=[
        pltpu.VMEM((SEQ,), jnp.int32),
    ]
    + [pltpu.VMEM((K, DIM), jnp.float32)] * NBUF
    + [pltpu.SemaphoreType.DMA] * NBUF
    + [pltpu.SemaphoreType.DMA] * NBUF,
)
def _pos_embed_sc(tok_hbm, w_hbm, out_hbm, row_v, *rest):
    bufs = rest[:NBUF]
    gsem = rest[NBUF:2 * NBUF]
    osem = rest[2 * NBUF:3 * NBUF]
    cid = lax.axis_index("c")
    sid = lax.axis_index("s")
    wid = cid * NSUB + sid
    b = wid // ROWCHUNKS
    cpos = wid % ROWCHUNKS

    pltpu.sync_copy(tok_hbm.at[b], row_v)

    def count_body(i, acc):
        for u in range(8):
            v = row_v[pl.ds(i * 128 + u * 16, 16)]
            acc = acc + jnp.minimum(jnp.abs(v - PAD), 1)
        return acc

    acc = lax.fori_loop(0, cpos * (CHUNK // 128), count_body,
                        jnp.zeros((16,), jnp.int32))
    carry = jnp.sum(acc)

    base = cpos * CHUNK

    def compute_step(s, carry):
        v = row_v[pl.ds(base + s * 16, 16)]
        m = jnp.minimum(jnp.abs(v - PAD), 1)
        cum = jnp.cumsum(m) + carry
        pos = cum * m + PAD
        return pos, carry + jnp.sum(m)

    rbase = wid * CHUNK

    def gather(t, pos):
        return pltpu.async_copy(
            w_hbm.at[pos], bufs[t % NBUF], gsem[t % NBUF])

    def writeout(t):
        return pltpu.async_copy(
            bufs[t % NBUF], out_hbm.at[pl.ds(rbase + t * K, K)],
            osem[t % NBUF])

    gh = [None] * NSTEP
    oh = [None] * NSTEP
    for t in range(NFLY):
        pos, carry = compute_step(t, carry)
        gh[t] = gather(t, pos)
    for t in range(NSTEP):
        if t + NFLY < NSTEP:
            pos, carry = compute_step(t + NFLY, carry)
        gh[t].wait()
        oh[t] = writeout(t)
        if t + NFLY < NSTEP:
            tb = t + NFLY - NBUF
            if tb >= 0:
                oh[tb].wait()
            gh[t + NFLY] = gather(t + NFLY, pos)
    for t in range(max(0, NSTEP - NBUF), NSTEP):
        oh[t].wait()


def kernel(inputs, weights):
    tok = inputs.astype(jnp.int32)
    out = _pos_embed_sc(tok, weights)
    return lax.stop_gradient(out.reshape(BATCH, SEQ, DIM))
